# Initial kernel scaffold; baseline (speedup 1.0000x reference)
import functools

import jax
import jax.numpy as jnp
from jax import lax
from jax.experimental import pallas as pl
from jax.experimental.pallas import tpu as pltpu
from jax.experimental.pallas import tpu_sc as plsc

_B, _C, _H, _W = 4, 3, 1024, 2048
_F, _GRID = 80, 128
_N = _F * _GRID * _GRID
_BC = _B * _C
_CP = 16
_NW = 32
_SPW = _N // _NW
_CH = 128
_UVBLK = 4096
_NCH_PER_BLK = _UVBLK // _CH
_NBLK = _SPW // _UVBLK


def _sc_body(xt_hbm, u_hbm, v_hbm, out_hbm,
             u_v, v_v, du_v, dv_v, idx_v, gat_v, out_v, sem):
    wid = lax.axis_index("s") * 2 + lax.axis_index("c")
    base0 = wid * _SPW
    iota16 = lax.iota(jnp.int32, 16)

    def blk_body(blk, _):
        bbase = base0 + blk * _UVBLK
        pltpu.sync_copy(u_hbm.at[pl.ds(bbase, _UVBLK)], u_v)
        pltpu.sync_copy(v_hbm.at[pl.ds(bbase, _UVBLK)], v_v)

        def chunk_body(ci, _):
            off = ci * _CH
            for g in range(8):
                s = off + g * 16
                u16 = u_v[pl.ds(s, 16)]
                v16 = v_v[pl.ds(s, 16)]
                u0 = jnp.minimum(u16.astype(jnp.int32), _W - 2)
                v0 = jnp.minimum(v16.astype(jnp.int32), _H - 2)
                du = u16 - u0.astype(jnp.float32)
                dv = v16 - v0.astype(jnp.float32)
                i00 = v0 * _W + u0
                idx_v[0, pl.ds(g * 16, 16)] = i00
                idx_v[1, pl.ds(g * 16, 16)] = i00 + 1
                idx_v[2, pl.ds(g * 16, 16)] = i00 + _W
                idx_v[3, pl.ds(g * 16, 16)] = i00 + _W + 1
                du_v[pl.ds(g * 16, 16)] = du
                dv_v[pl.ds(g * 16, 16)] = dv
            copies = [
                pltpu.async_copy(xt_hbm.at[idx_v.at[k]], gat_v.at[k], sem)
                for k in range(4)
            ]
            for cp in copies:
                cp.wait()
            for g in range(8):
                du = du_v[pl.ds(g * 16, 16)]
                dv = dv_v[pl.ds(g * 16, 16)]
                svec = iota16 + (g * 16)
                for c in range(_BC):
                    cs = jnp.full((16,), c, jnp.int32)
                    p00 = plsc.load_gather(gat_v.at[0], [svec, cs])
                    p01 = plsc.load_gather(gat_v.at[1], [svec, cs])
                    p10 = plsc.load_gather(gat_v.at[2], [svec, cs])
                    p11 = plsc.load_gather(gat_v.at[3], [svec, cs])
                    a = p00 + du * (p01 - p00)
                    b = p10 + du * (p11 - p10)
                    out_v[c, pl.ds(off + g * 16, 16)] = a + dv * (b - a)
            return 0

        lax.fori_loop(0, _NCH_PER_BLK, chunk_body, 0)
        for c in range(_BC):
            pltpu.sync_copy(out_v.at[c], out_hbm.at[c, pl.ds(bbase, _UVBLK)])
        return 0

    lax.fori_loop(0, _NBLK, blk_body, 0)


_sc_kernel = functools.partial(
    pl.kernel,
    out_type=jax.ShapeDtypeStruct((_BC, _N), jnp.float32),
    mesh=plsc.VectorSubcoreMesh(core_axis_name="c", subcore_axis_name="s"),
    scratch_types=[
        pltpu.VMEM((_UVBLK,), jnp.float32),
        pltpu.VMEM((_UVBLK,), jnp.float32),
        pltpu.VMEM((_CH,), jnp.float32),
        pltpu.VMEM((_CH,), jnp.float32),
        pltpu.VMEM((4, _CH), jnp.int32),
        pltpu.VMEM((4, _CH, _CP), jnp.float32),
        pltpu.VMEM((_BC, _UVBLK), jnp.float32),
        pltpu.SemaphoreType.DMA,
    ],
)(_sc_body)


def kernel(x, sample_map):
    xt = jnp.pad(x.reshape(_BC, _H * _W).T, ((0, 0), (0, _CP - _BC)))
    sm = sample_map.reshape(_N, 2)
    out = _sc_kernel(xt, sm[:, 0], sm[:, 1])
    return out.reshape(_B, _C, _F, _GRID, _GRID)


# speedup vs baseline: 36.7888x; 36.7888x over previous
import functools

import jax
import jax.numpy as jnp
from jax import lax
from jax.experimental import pallas as pl
from jax.experimental.pallas import tpu as pltpu
from jax.experimental.pallas import tpu_sc as plsc

_B, _C, _H, _W = 4, 3, 1024, 2048
_F, _GRID = 80, 128
_N = _F * _GRID * _GRID
_BC = _B * _C
_CP = 16
_NW = 32
_SPW = _N // _NW
_CH = 128
_UVBLK = 4096
_NCH_PER_BLK = _UVBLK // _CH
_NBLK = _SPW // _UVBLK


def _sc_body(xt_hbm, u_hbm, v_hbm, out_hbm,
             u_v, v_v, du_v, dv_v, idx_v, gat0, gat1, gat2, gat3, out_v, sem):
    gats = (gat0, gat1, gat2, gat3)
    wid = lax.axis_index("s") * 2 + lax.axis_index("c")
    base0 = wid * _SPW
    iota16 = lax.iota(jnp.int32, 16)

    def blk_body(blk, _):
        bbase = base0 + blk * _UVBLK
        pltpu.sync_copy(u_hbm.at[pl.ds(bbase, _UVBLK)], u_v)
        pltpu.sync_copy(v_hbm.at[pl.ds(bbase, _UVBLK)], v_v)

        def chunk_body(ci, _):
            off = ci * _CH
            for g in range(8):
                s = off + g * 16
                u16 = u_v[pl.ds(s, 16)]
                v16 = v_v[pl.ds(s, 16)]
                u0 = jnp.minimum(u16.astype(jnp.int32), _W - 2)
                v0 = jnp.minimum(v16.astype(jnp.int32), _H - 2)
                du = u16 - u0.astype(jnp.float32)
                dv = v16 - v0.astype(jnp.float32)
                i00 = v0 * _W + u0
                idx_v[0, pl.ds(g * 16, 16)] = i00
                idx_v[1, pl.ds(g * 16, 16)] = i00 + 1
                idx_v[2, pl.ds(g * 16, 16)] = i00 + _W
                idx_v[3, pl.ds(g * 16, 16)] = i00 + _W + 1
                du_v[pl.ds(g * 16, 16)] = du
                dv_v[pl.ds(g * 16, 16)] = dv
            copies = [
                pltpu.async_copy(xt_hbm.at[idx_v.at[k]], gats[k], sem)
                for k in range(4)
            ]
            for cp in copies:
                cp.wait()
            for g in range(8):
                du = du_v[pl.ds(g * 16, 16)]
                dv = dv_v[pl.ds(g * 16, 16)]
                svec = iota16 + (g * 16)
                for c in range(_BC):
                    cs = jnp.full((16,), c, jnp.int32)
                    p00 = plsc.load_gather(gat0, [svec, cs])
                    p01 = plsc.load_gather(gat1, [svec, cs])
                    p10 = plsc.load_gather(gat2, [svec, cs])
                    p11 = plsc.load_gather(gat3, [svec, cs])
                    a = p00 + du * (p01 - p00)
                    b = p10 + du * (p11 - p10)
                    out_v[c, pl.ds(off + g * 16, 16)] = a + dv * (b - a)
            return 0

        lax.fori_loop(0, _NCH_PER_BLK, chunk_body, 0)
        for c in range(_BC):
            pltpu.sync_copy(out_v.at[c], out_hbm.at[c, pl.ds(bbase, _UVBLK)])
        return 0

    lax.fori_loop(0, _NBLK, blk_body, 0)


_sc_kernel = functools.partial(
    pl.kernel,
    out_type=jax.ShapeDtypeStruct((_BC, _N), jnp.float32),
    mesh=plsc.VectorSubcoreMesh(core_axis_name="c", subcore_axis_name="s"),
    compiler_params=pltpu.CompilerParams(
        needs_layout_passes=False, use_tc_tiling_on_sc=False),
    scratch_types=[
        pltpu.VMEM((_UVBLK,), jnp.float32),
        pltpu.VMEM((_UVBLK,), jnp.float32),
        pltpu.VMEM((_CH,), jnp.float32),
        pltpu.VMEM((_CH,), jnp.float32),
        pltpu.VMEM((4, _CH), jnp.int32),
        pltpu.VMEM((_CH, _CP), jnp.float32),
        pltpu.VMEM((_CH, _CP), jnp.float32),
        pltpu.VMEM((_CH, _CP), jnp.float32),
        pltpu.VMEM((_CH, _CP), jnp.float32),
        pltpu.VMEM((_BC, _UVBLK), jnp.float32),
        pltpu.SemaphoreType.DMA,
    ],
)(_sc_body)


def kernel(x, sample_map):
    xt = jnp.pad(x.reshape(_BC, _H * _W).T, ((0, 0), (0, _CP - _BC)))
    sm = sample_map.reshape(_N, 2)
    out = _sc_kernel(xt, sm[:, 0], sm[:, 1])
    return out.reshape(_B, _C, _F, _GRID, _GRID)
